# Initial kernel scaffold; baseline (speedup 1.0000x reference)
import jax
import jax.numpy as jnp
from jax.experimental import pallas as pl


def kernel(hidden_states, input_ids, w1, w2):
    raise NotImplementedError("write your pallas kernel here")


# speedup vs baseline: 1.0787x; 1.0787x over previous
import functools

import jax
import jax.numpy as jnp
import numpy as np
from jax import lax
from jax.experimental import pallas as pl
from jax.experimental.pallas import tpu as pltpu
from jax.experimental.pallas import tpu_sc as plsc

E = 64
D = 1024
F = 2048
CAP_FACTOR = 1.25
NC, NS = 2, 16
NW = NC * NS


def _make_dispatch(T, slots, chunk):
  per_w = slots // NW
  n_ch = per_w // chunk
  mesh = plsc.VectorSubcoreMesh(core_axis_name="c", subcore_axis_name="s")

  @functools.partial(
      pl.kernel, mesh=mesh,
      out_type=jax.ShapeDtypeStruct((slots, D), jnp.float32),
      scratch_types=[
          pltpu.VMEM((per_w,), jnp.int32),
          pltpu.VMEM((chunk, D), jnp.float32),
          pltpu.VMEM((chunk, D), jnp.float32),
          pltpu.SemaphoreType.DMA,
          pltpu.SemaphoreType.DMA,
      ],
  )
  def dispatch(x_hbm, idx_hbm, buf_hbm, idx_v, rows0, rows1, sem0, sem1):
    wid = lax.axis_index("s") * NC + lax.axis_index("c")
    base = wid * per_w
    pltpu.sync_copy(idx_hbm.at[pl.ds(base, per_w)], idx_v)
    rows = (rows0, rows1)
    sems = (sem0, sem1)
    cps = [None, None]
    cps[0] = pltpu.async_copy(x_hbm.at[idx_v.at[pl.ds(0, chunk)]], rows0, sem0)
    for k in range(n_ch):
      if k + 1 < n_ch:
        cps[(k + 1) % 2] = pltpu.async_copy(
            x_hbm.at[idx_v.at[pl.ds((k + 1) * chunk, chunk)]],
            rows[(k + 1) % 2], sems[(k + 1) % 2])
      cps[k % 2].wait()
      pltpu.sync_copy(rows[k % 2], buf_hbm.at[pl.ds(base + k * chunk, chunk)])

  return dispatch


def _ffn_body(C, C2, buf_ref, w1_ref, w2_ref, y_ref):
  f = pl.program_id(1)
  b = buf_ref[0]
  h = jnp.maximum(
      lax.dot_general(b, w1_ref[0], (((1,), (0,)), ((), ())),
                      preferred_element_type=jnp.float32), 0.0)
  yp = lax.dot_general(h, w2_ref[0], (((1,), (0,)), ((), ())),
                       preferred_element_type=jnp.float32)
  rowmask = (lax.broadcasted_iota(jnp.int32, (C2, 1), 0) < C).astype(yp.dtype)
  yp = yp * rowmask

  @pl.when(f == 0)
  def _():
    y_ref[0] = yp

  @pl.when(f != 0)
  def _():
    y_ref[0] = y_ref[0] + yp


def _make_ffn(C, C2, fblk):
  nf = F // fblk
  return pl.pallas_call(
      functools.partial(_ffn_body, C, C2),
      grid=(E, nf),
      in_specs=[
          pl.BlockSpec((1, C2, D), lambda e, f: (e, 0, 0)),
          pl.BlockSpec((1, D, fblk), lambda e, f: (e, 0, f)),
          pl.BlockSpec((1, fblk, D), lambda e, f: (e, f, 0)),
      ],
      out_specs=pl.BlockSpec((1, C2, D), lambda e, f: (e, 0, 0)),
      out_shape=jax.ShapeDtypeStruct((E, C2, D), jnp.float32),
      compiler_params=pltpu.CompilerParams(
          dimension_semantics=("arbitrary", "arbitrary")),
  )


def _make_combine(T, slots, chunk):
  per_w = T // NW
  n_ch = per_w // chunk
  mesh = plsc.VectorSubcoreMesh(core_axis_name="c", subcore_axis_name="s")

  @functools.partial(
      pl.kernel, mesh=mesh,
      out_type=jax.ShapeDtypeStruct((T, D), jnp.float32),
      scratch_types=[
          pltpu.VMEM((per_w,), jnp.int32),
          pltpu.VMEM((chunk, D), jnp.float32),
          pltpu.VMEM((chunk, D), jnp.float32),
          pltpu.SemaphoreType.DMA,
          pltpu.SemaphoreType.DMA,
      ],
  )
  def combine(y_hbm, dest_hbm, out_hbm, idx_v, rows0, rows1, sem0, sem1):
    wid = lax.axis_index("s") * NC + lax.axis_index("c")
    base = wid * per_w
    pltpu.sync_copy(dest_hbm.at[pl.ds(base, per_w)], idx_v)
    rows = (rows0, rows1)
    sems = (sem0, sem1)
    cps = [None, None]
    cps[0] = pltpu.async_copy(y_hbm.at[idx_v.at[pl.ds(0, chunk)]], rows0, sem0)
    for k in range(n_ch):
      if k + 1 < n_ch:
        cps[(k + 1) % 2] = pltpu.async_copy(
            y_hbm.at[idx_v.at[pl.ds((k + 1) * chunk, chunk)]],
            rows[(k + 1) % 2], sems[(k + 1) % 2])
      cps[k % 2].wait()
      pltpu.sync_copy(rows[k % 2], out_hbm.at[pl.ds(base + k * chunk, chunk)])

  return combine


@jax.jit
def kernel(hidden_states, input_ids, w1, w2):
  B, S, _ = hidden_states.shape
  T = B * S
  C = int(np.ceil(T / E * CAP_FACTOR))
  C2 = C + 16
  slots = E * C2

  x = hidden_states.reshape(T, D)
  ids = input_ids.reshape(T).astype(jnp.int32)

  expert = jnp.mod(ids, E)
  oh = (expert[:, None] == jnp.arange(E, dtype=jnp.int32)[None, :]).astype(
      jnp.int32)
  cum = jnp.cumsum(oh, axis=0)
  slot = jnp.take_along_axis(cum, expert[:, None], axis=1)[:, 0] - 1
  keep = slot < C
  tgt = jnp.where(keep, expert * C2 + slot, slots)
  idx_d = jnp.zeros((slots,), jnp.int32).at[tgt].set(
      jnp.arange(T, dtype=jnp.int32), mode="drop")
  dest = jnp.where(keep, expert * C2 + slot, expert * C2 + C).astype(jnp.int32)

  buf = _make_dispatch(T, slots, 48)(x, idx_d)
  y = _make_ffn(C, C2, 1024)(buf.reshape(E, C2, D), w1, w2)
  out = _make_combine(T, slots, 32)(y.reshape(slots, D), dest)
  return out.reshape(B, S, D)


# speedup vs baseline: 1.3375x; 1.2399x over previous
import functools

import jax
import jax.numpy as jnp
import numpy as np
from jax import lax
from jax.experimental import pallas as pl
from jax.experimental.pallas import tpu as pltpu
from jax.experimental.pallas import tpu_sc as plsc

E = 64
D = 1024
F = 2048
CAP_FACTOR = 1.25
NC, NS = 2, 16
NW = NC * NS


def _make_dispatch(T, slots, chunk):
  per_w = slots // NW
  n_ch = per_w // chunk
  mesh = plsc.VectorSubcoreMesh(core_axis_name="c", subcore_axis_name="s")

  @functools.partial(
      pl.kernel, mesh=mesh,
      out_type=jax.ShapeDtypeStruct((slots, D), jnp.float32),
      scratch_types=[
          pltpu.VMEM((per_w,), jnp.int32),
          pltpu.VMEM((chunk, D), jnp.float32),
          pltpu.VMEM((chunk, D), jnp.float32),
          pltpu.SemaphoreType.DMA,
          pltpu.SemaphoreType.DMA,
      ],
  )
  def dispatch(x_hbm, idx_hbm, buf_hbm, idx_v, rows0, rows1, sem0, sem1):
    wid = lax.axis_index("s") * NC + lax.axis_index("c")
    base = wid * per_w
    pltpu.sync_copy(idx_hbm.at[pl.ds(base, per_w)], idx_v)
    rows = (rows0, rows1)
    sems = (sem0, sem1)
    cps = [None, None]
    cps[0] = pltpu.async_copy(x_hbm.at[idx_v.at[pl.ds(0, chunk)]], rows0, sem0)
    for k in range(n_ch):
      if k + 1 < n_ch:
        cps[(k + 1) % 2] = pltpu.async_copy(
            x_hbm.at[idx_v.at[pl.ds((k + 1) * chunk, chunk)]],
            rows[(k + 1) % 2], sems[(k + 1) % 2])
      cps[k % 2].wait()
      pltpu.sync_copy(rows[k % 2], buf_hbm.at[pl.ds(base + k * chunk, chunk)])

  return dispatch


def _ffn_body(C, C2, buf_ref, w1_ref, w2_ref, y_ref):
  f = pl.program_id(1)
  b = buf_ref[0]
  h = jnp.maximum(
      lax.dot_general(b, w1_ref[0], (((1,), (0,)), ((), ())),
                      preferred_element_type=jnp.float32), 0.0)
  yp = lax.dot_general(h, w2_ref[0], (((1,), (0,)), ((), ())),
                       preferred_element_type=jnp.float32)
  rowmask = (lax.broadcasted_iota(jnp.int32, (C2, 1), 0) < C).astype(yp.dtype)
  yp = yp * rowmask

  @pl.when(f == 0)
  def _():
    y_ref[0] = yp

  @pl.when(f != 0)
  def _():
    y_ref[0] = y_ref[0] + yp


def _make_ffn(C, C2, fblk):
  nf = F // fblk
  return pl.pallas_call(
      functools.partial(_ffn_body, C, C2),
      grid=(E, nf),
      in_specs=[
          pl.BlockSpec((1, C2, D), lambda e, f: (e, 0, 0)),
          pl.BlockSpec((1, D, fblk), lambda e, f: (e, 0, f)),
          pl.BlockSpec((1, fblk, D), lambda e, f: (e, f, 0)),
      ],
      out_specs=pl.BlockSpec((1, C2, D), lambda e, f: (e, 0, 0)),
      out_shape=jax.ShapeDtypeStruct((E, C2, D), jnp.float32),
      compiler_params=pltpu.CompilerParams(
          dimension_semantics=("arbitrary", "arbitrary")),
  )


def _make_combine(T, slots, chunk):
  per_w = T // NW
  n_ch = per_w // chunk
  mesh = plsc.VectorSubcoreMesh(core_axis_name="c", subcore_axis_name="s")

  @functools.partial(
      pl.kernel, mesh=mesh,
      out_type=jax.ShapeDtypeStruct((T, D), jnp.float32),
      scratch_types=[
          pltpu.VMEM((per_w,), jnp.int32),
          pltpu.VMEM((chunk, D), jnp.float32),
          pltpu.VMEM((chunk, D), jnp.float32),
          pltpu.SemaphoreType.DMA,
          pltpu.SemaphoreType.DMA,
      ],
  )
  def combine(y_hbm, dest_hbm, out_hbm, idx_v, rows0, rows1, sem0, sem1):
    wid = lax.axis_index("s") * NC + lax.axis_index("c")
    base = wid * per_w
    pltpu.sync_copy(dest_hbm.at[pl.ds(base, per_w)], idx_v)
    rows = (rows0, rows1)
    sems = (sem0, sem1)
    cps = [None, None]
    cps[0] = pltpu.async_copy(y_hbm.at[idx_v.at[pl.ds(0, chunk)]], rows0, sem0)
    for k in range(n_ch):
      if k + 1 < n_ch:
        cps[(k + 1) % 2] = pltpu.async_copy(
            y_hbm.at[idx_v.at[pl.ds((k + 1) * chunk, chunk)]],
            rows[(k + 1) % 2], sems[(k + 1) % 2])
      cps[k % 2].wait()
      pltpu.sync_copy(rows[k % 2], out_hbm.at[pl.ds(base + k * chunk, chunk)])

  return combine


@jax.jit
def kernel(hidden_states, input_ids, w1, w2):
  B, S, _ = hidden_states.shape
  T = B * S
  C = int(np.ceil(T / E * CAP_FACTOR))
  C2 = C + 16
  slots = E * C2

  x = hidden_states.reshape(T, D)
  ids = input_ids.reshape(T).astype(jnp.int32)

  expert = jnp.mod(ids, E)
  oh = (expert[:, None] == jnp.arange(E, dtype=jnp.int32)[None, :]).astype(
      jnp.int32)
  cum = jnp.cumsum(oh, axis=0)
  slot = jnp.take_along_axis(cum, expert[:, None], axis=1)[:, 0] - 1
  keep = slot < C
  tgt = jnp.where(keep, expert * C2 + slot, slots)
  idx_d = (jnp.arange(slots, dtype=jnp.int32) % T).at[tgt].set(
      jnp.arange(T, dtype=jnp.int32), mode="drop")
  dest = jnp.where(keep, expert * C2 + slot, expert * C2 + C).astype(jnp.int32)

  buf = _make_dispatch(T, slots, 48)(x, idx_d)
  y = _make_ffn(C, C2, 1024)(buf.reshape(E, C2, D), w1, w2)
  out = _make_combine(T, slots, 32)(y.reshape(slots, D), dest)
  return out.reshape(B, S, D)
